# Initial kernel scaffold; baseline (speedup 1.0000x reference)
import jax
import jax.numpy as jnp
from jax.experimental import pallas as pl


def kernel(inputFeatures, edge_index, W1, b1, W2, b2, W3, b3, Wl1, bl1, Wl2, bl2, Wl3, bl3):
    raise NotImplementedError("write your pallas kernel here")


# speedup vs baseline: 11.8857x; 11.8857x over previous
import functools

import jax
import jax.numpy as jnp
from jax import lax
from jax.experimental import pallas as pl
from jax.experimental.pallas import tpu as pltpu
from jax.experimental.pallas import tpu_sc as plsc

N = 10000
E = 320000
NT = 32
CH = 128
NCHUNK = 80
EPT = CH * NCHUNK
EPAD = EPT * NT
NPAD = 10112
ROWS_PT = NPAD // 16
DUMMY = N
BLK = 1000
GRID = N // BLK


def _sc_agg(dp):
  mesh = plsc.VectorSubcoreMesh(core_axis_name="c", subcore_axis_name="s")

  @functools.partial(
      pl.kernel,
      mesh=mesh,
      out_type=jax.ShapeDtypeStruct((2, NPAD, dp), jnp.float32),
      scratch_types=[
          pltpu.VMEM((NCHUNK, CH), jnp.int32),
          pltpu.VMEM((NCHUNK, CH), jnp.int32),
          pltpu.VMEM((CH, dp), jnp.float32),
          pltpu.VMEM_SHARED((NPAD, dp), jnp.float32),
          pltpu.SemaphoreType.DMA,
      ],
      compiler_params=pltpu.CompilerParams(use_tc_tiling_on_sc=False),
  )
  def k(hp, src3, dst3, zeros_hbm, out, src_v, dst_v, rows_v, acc, sem):
    c = lax.axis_index("c")
    s = lax.axis_index("s")
    wid = c * 16 + s
    row0 = s * ROWS_PT
    pltpu.sync_copy(src3.at[wid], src_v)
    pltpu.sync_copy(dst3.at[wid], dst_v)
    pltpu.sync_copy(zeros_hbm, acc.at[pl.ds(row0, ROWS_PT)])
    plsc.subcore_barrier()

    def step(j, carry):
      pltpu.async_copy(hp.at[src_v.at[j]], rows_v, sem).wait()
      pltpu.sync_copy(rows_v, acc.at[dst_v.at[j]], add=True)
      return carry

    lax.fori_loop(0, NCHUNK, step, 0)
    plsc.subcore_barrier()
    pltpu.sync_copy(acc.at[pl.ds(row0, ROWS_PT)],
                    out.at[c, pl.ds(row0, ROWS_PT)])

  return k


def _sc_deg():
  mesh = plsc.VectorSubcoreMesh(core_axis_name="c", subcore_axis_name="s")

  @functools.partial(
      pl.kernel,
      mesh=mesh,
      out_type=jax.ShapeDtypeStruct((2, NPAD, 16), jnp.float32),
      scratch_types=[
          pltpu.VMEM((NCHUNK, CH), jnp.int32),
          pltpu.VMEM((CH, 16), jnp.float32),
          pltpu.VMEM_SHARED((NPAD, 16), jnp.float32),
      ],
      compiler_params=pltpu.CompilerParams(use_tc_tiling_on_sc=False),
  )
  def k(dst3, ones_hbm, zeros_hbm, out, dst_v, ones_v, acc):
    c = lax.axis_index("c")
    s = lax.axis_index("s")
    wid = c * 16 + s
    row0 = s * ROWS_PT
    pltpu.sync_copy(dst3.at[wid], dst_v)
    pltpu.sync_copy(ones_hbm, ones_v)
    pltpu.sync_copy(zeros_hbm, acc.at[pl.ds(row0, ROWS_PT)])
    plsc.subcore_barrier()

    def step(j, carry):
      pltpu.sync_copy(ones_v, acc.at[dst_v.at[j]], add=True)
      return carry

    lax.fori_loop(0, NCHUNK, step, 0)
    plsc.subcore_barrier()
    pltpu.sync_copy(acc.at[pl.ds(row0, ROWS_PT)],
                    out.at[c, pl.ds(row0, ROWS_PT)])

  return k


def _dinv_of(deg_ref):
  d = deg_ref[...]
  return lax.rsqrt(1.0 + d[0, :, :1] + d[1, :, :1])


def _tc_first(x, w, deg):
  def body(x_ref, w_ref, deg_ref, o_ref):
    dinv = _dinv_of(deg_ref)
    o_ref[...] = jnp.dot(x_ref[...], w_ref[...],
                         preferred_element_type=jnp.float32) * dinv

  return pl.pallas_call(
      body,
      grid=(GRID,),
      in_specs=[
          pl.BlockSpec((BLK, 128), lambda i: (i, 0)),
          pl.BlockSpec((128, 128), lambda i: (0, 0)),
          pl.BlockSpec((2, BLK, 16), lambda i: (0, i, 0)),
      ],
      out_specs=pl.BlockSpec((BLK, 128), lambda i: (i, 0)),
      out_shape=jax.ShapeDtypeStruct((N, 128), jnp.float32),
  )(x, w, deg)


def _tc_mid(agg, hp, deg, w, b, dp_in, dp_out):
  def body(agg_ref, hp_ref, deg_ref, w_ref, b_ref, o_ref):
    dinv = _dinv_of(deg_ref)
    ag = agg_ref[...]
    xv = jnp.maximum(dinv * (ag[0] + ag[1] + hp_ref[...]) + b_ref[...], 0.0)
    o_ref[...] = jnp.dot(xv, w_ref[...],
                         preferred_element_type=jnp.float32) * dinv

  return pl.pallas_call(
      body,
      grid=(GRID,),
      in_specs=[
          pl.BlockSpec((2, BLK, dp_in), lambda i: (0, i, 0)),
          pl.BlockSpec((BLK, dp_in), lambda i: (i, 0)),
          pl.BlockSpec((2, BLK, 16), lambda i: (0, i, 0)),
          pl.BlockSpec((dp_in, dp_out), lambda i: (0, 0)),
          pl.BlockSpec((1, dp_in), lambda i: (0, 0)),
      ],
      out_specs=pl.BlockSpec((BLK, dp_out), lambda i: (i, 0)),
      out_shape=jax.ShapeDtypeStruct((N, dp_out), jnp.float32),
  )(agg, hp, deg, w, b)


def _tc_final(agg, hp, deg, b3, wl1, bl1, wl2, bl2, wl3, bl3):
  def body(agg_ref, hp_ref, deg_ref, b3_ref, w1_ref, b1_ref, w2_ref, b2_ref,
           w3_ref, b3h_ref, o_ref):
    dinv = _dinv_of(deg_ref)
    ag = agg_ref[...]
    y = jnp.maximum(dinv * (ag[0] + ag[1] + hp_ref[...]) + b3_ref[...], 0.0)
    y = jnp.maximum(jnp.dot(y, w1_ref[...],
                            preferred_element_type=jnp.float32) + b1_ref[...],
                    0.0)
    y = jnp.maximum(jnp.dot(y, w2_ref[...],
                            preferred_element_type=jnp.float32) + b2_ref[...],
                    0.0)
    y = jnp.maximum(jnp.dot(y, w3_ref[...],
                            preferred_element_type=jnp.float32) + b3h_ref[...],
                    0.0)
    o_ref[...] = y

  return pl.pallas_call(
      body,
      grid=(GRID,),
      in_specs=[
          pl.BlockSpec((2, BLK, 32), lambda i: (0, i, 0)),
          pl.BlockSpec((BLK, 32), lambda i: (i, 0)),
          pl.BlockSpec((2, BLK, 16), lambda i: (0, i, 0)),
          pl.BlockSpec((1, 32), lambda i: (0, 0)),
          pl.BlockSpec((32, 32), lambda i: (0, 0)),
          pl.BlockSpec((1, 32), lambda i: (0, 0)),
          pl.BlockSpec((32, 16), lambda i: (0, 0)),
          pl.BlockSpec((1, 16), lambda i: (0, 0)),
          pl.BlockSpec((16, 8), lambda i: (0, 0)),
          pl.BlockSpec((1, 8), lambda i: (0, 0)),
      ],
      out_specs=pl.BlockSpec((BLK, 8), lambda i: (i, 0)),
      out_shape=jax.ShapeDtypeStruct((N, 8), jnp.float32),
  )(agg, hp, deg, b3, wl1, bl1, wl2, bl2, wl3, bl3)


def _pad2(w, r, c):
  return jnp.zeros((r, c), w.dtype).at[:w.shape[0], :w.shape[1]].set(w)


def _pad1(b, c):
  return jnp.zeros((1, c), b.dtype).at[0, :b.shape[0]].set(b)


def kernel(inputFeatures, edge_index, W1, b1, W2, b2, W3, b3,
           Wl1, bl1, Wl2, bl2, Wl3, bl3):
  src = edge_index[0]
  dst = edge_index[1]
  npad_e = EPAD - E
  src3 = jnp.concatenate(
      [src, jnp.zeros((npad_e,), src.dtype)]).reshape(NT, NCHUNK, CH)
  dst3 = jnp.concatenate(
      [dst, jnp.full((npad_e,), DUMMY, dst.dtype)]).reshape(NT, NCHUNK, CH)
  ones16 = jnp.ones((CH, 16), jnp.float32)
  z16 = jnp.zeros((ROWS_PT, 16), jnp.float32)
  z128 = jnp.zeros((ROWS_PT, 128), jnp.float32)
  z64 = jnp.zeros((ROWS_PT, 64), jnp.float32)
  z32 = jnp.zeros((ROWS_PT, 32), jnp.float32)

  deg = _sc_deg()(dst3, ones16, z16)
  hp1 = _tc_first(inputFeatures, _pad2(W1, 128, 128), deg)
  agg1 = _sc_agg(128)(hp1, src3, dst3, z128)
  hp2 = _tc_mid(agg1, hp1, deg, _pad2(W2, 128, 64), _pad1(b1, 128), 128, 64)
  agg2 = _sc_agg(64)(hp2, src3, dst3, z64)
  hp3 = _tc_mid(agg2, hp2, deg, _pad2(W3, 64, 32), _pad1(b2, 64), 64, 32)
  agg3 = _sc_agg(32)(hp3, src3, dst3, z32)
  y = _tc_final(agg3, hp3, deg, _pad1(b3, 32),
                _pad2(Wl1, 32, 32), _pad1(bl1, 32),
                _pad2(Wl2, 32, 16), _pad1(bl2, 16),
                _pad2(Wl3, 16, 8), _pad1(bl3, 8))
  return y[:, :1]


# speedup vs baseline: 18.7987x; 1.5816x over previous
import functools

import jax
import jax.numpy as jnp
from jax import lax
from jax.experimental import pallas as pl
from jax.experimental.pallas import tpu as pltpu
from jax.experimental.pallas import tpu_sc as plsc

N = 10000
E = 320000
NPAD = 10112
ROWS_PT = NPAD // 16
DUMMY = N
BLK = 1000
GRID = N // BLK

CH1 = 80
NCH1 = 128
EPAD1 = 32 * NCH1 * CH1

CH2 = 128
NCH2 = 158
EPAD2 = 16 * NCH2 * CH2

_SC_PARAMS = pltpu.CompilerParams(use_tc_tiling_on_sc=False)


def _mesh():
  return plsc.VectorSubcoreMesh(core_axis_name="c", subcore_axis_name="s")


def _sc_deg():
  @functools.partial(
      pl.kernel,
      mesh=_mesh(),
      out_type=jax.ShapeDtypeStruct((2, NPAD, 16), jnp.float32),
      scratch_types=[
          pltpu.VMEM((NCH1, CH1), jnp.int32),
          pltpu.VMEM((CH1, 16), jnp.float32),
          pltpu.VMEM_SHARED((NPAD, 16), jnp.float32),
      ],
      compiler_params=_SC_PARAMS,
  )
  def k(dst3, ones_hbm, zeros_hbm, out, dst_v, ones_v, acc):
    c = lax.axis_index("c")
    s = lax.axis_index("s")
    wid = c * 16 + s
    row0 = s * ROWS_PT
    pltpu.sync_copy(dst3.at[wid], dst_v)
    pltpu.sync_copy(ones_hbm, ones_v)
    pltpu.sync_copy(zeros_hbm, acc.at[pl.ds(row0, ROWS_PT)])
    plsc.subcore_barrier()

    def step(j, carry):
      pltpu.sync_copy(ones_v, acc.at[dst_v.at[j]], add=True)
      return carry

    lax.fori_loop(0, NCH1, step, 0)
    plsc.subcore_barrier()
    pltpu.sync_copy(acc.at[pl.ds(row0, ROWS_PT)],
                    out.at[c, pl.ds(row0, ROWS_PT)])

  return k


def _sc_agg_l1():
  @functools.partial(
      pl.kernel,
      mesh=_mesh(),
      out_type=jax.ShapeDtypeStruct((2, NPAD, 128), jnp.float32),
      scratch_types=[
          pltpu.VMEM((NCH1, CH1), jnp.int32),
          pltpu.VMEM((NCH1, CH1), jnp.int32),
          pltpu.VMEM((2, CH1, 128), jnp.float32),
          pltpu.VMEM_SHARED((NPAD, 128), jnp.float32),
          pltpu.SemaphoreType.DMA,
          pltpu.SemaphoreType.DMA,
      ],
      compiler_params=_SC_PARAMS,
  )
  def k(hp, src3, dst3, zeros_hbm, out, src_v, dst_v, rows_v, acc,
        sem0, sem1):
    c = lax.axis_index("c")
    s = lax.axis_index("s")
    wid = c * 16 + s
    row0 = s * ROWS_PT
    pltpu.sync_copy(src3.at[wid], src_v)
    pltpu.sync_copy(dst3.at[wid], dst_v)
    pltpu.sync_copy(zeros_hbm, acc.at[pl.ds(row0, ROWS_PT)])
    plsc.subcore_barrier()

    sems = (sem0, sem1)

    def gather(j, b):
      return pltpu.make_async_copy(hp.at[src_v.at[j]], rows_v.at[b], sems[b])

    gather(0, 0).start()
    gather(1, 1).start()

    def step(jj, carry):
      for b in range(2):
        j = jj * 2 + b
        gather(j, b).wait()
        pltpu.sync_copy(rows_v.at[b], acc.at[dst_v.at[j]], add=True)
        gather(j + 2, b).start()
      return carry

    lax.fori_loop(0, NCH1 // 2 - 1, step, 0)
    for b in range(2):
      j = NCH1 - 2 + b
      gather(j, b).wait()
      pltpu.sync_copy(rows_v.at[b], acc.at[dst_v.at[j]], add=True)

    plsc.subcore_barrier()
    pltpu.sync_copy(acc.at[pl.ds(row0, ROWS_PT)],
                    out.at[c, pl.ds(row0, ROWS_PT)])

  return k


def _sc_agg_cs(hdp):
  @functools.partial(
      pl.kernel,
      mesh=_mesh(),
      out_type=jax.ShapeDtypeStruct((2, NPAD, hdp), jnp.float32),
      scratch_types=[
          pltpu.VMEM((NCH2, CH2), jnp.int32),
          pltpu.VMEM((NCH2, CH2), jnp.int32),
          pltpu.VMEM((2, CH2, hdp), jnp.float32),
          pltpu.VMEM_SHARED((NPAD, hdp), jnp.float32),
          pltpu.VMEM_SHARED((NPAD, hdp), jnp.float32),
          pltpu.SemaphoreType.DMA,
          pltpu.SemaphoreType.DMA,
      ],
      compiler_params=_SC_PARAMS,
  )
  def k(hp3d, src3, dst3, zeros_hbm, out, src_v, dst_v, rows_v, hp_sp, acc,
        sem0, sem1):
    c = lax.axis_index("c")
    s = lax.axis_index("s")
    row0 = s * ROWS_PT
    pltpu.sync_copy(src3.at[s], src_v)
    pltpu.sync_copy(dst3.at[s], dst_v)
    pltpu.sync_copy(hp3d.at[c, pl.ds(row0, ROWS_PT)],
                    hp_sp.at[pl.ds(row0, ROWS_PT)])
    pltpu.sync_copy(zeros_hbm, acc.at[pl.ds(row0, ROWS_PT)])
    plsc.subcore_barrier()

    sems = (sem0, sem1)

    def gather(j, b):
      return pltpu.make_async_copy(hp_sp.at[src_v.at[j]], rows_v.at[b],
                                   sems[b])

    gather(0, 0).start()
    gather(1, 1).start()

    def step(jj, carry):
      for b in range(2):
        j = jj * 2 + b
        gather(j, b).wait()
        pltpu.sync_copy(rows_v.at[b], acc.at[dst_v.at[j]], add=True)
        gather(j + 2, b).start()
      return carry

    lax.fori_loop(0, NCH2 // 2 - 1, step, 0)
    for b in range(2):
      j = NCH2 - 2 + b
      gather(j, b).wait()
      pltpu.sync_copy(rows_v.at[b], acc.at[dst_v.at[j]], add=True)

    plsc.subcore_barrier()
    pltpu.sync_copy(acc.at[pl.ds(row0, ROWS_PT)],
                    out.at[c, pl.ds(row0, ROWS_PT)])

  return k


def _dinv_of(deg_ref):
  d = deg_ref[...]
  return lax.rsqrt(1.0 + d[0, :, :1] + d[1, :, :1])


def _tc_first(x, w, deg):
  def body(x_ref, w_ref, deg_ref, o_ref):
    dinv = _dinv_of(deg_ref)
    o_ref[...] = jnp.dot(x_ref[...], w_ref[...],
                         preferred_element_type=jnp.float32) * dinv

  return pl.pallas_call(
      body,
      grid=(GRID,),
      in_specs=[
          pl.BlockSpec((BLK, 128), lambda i: (i, 0)),
          pl.BlockSpec((128, 128), lambda i: (0, 0)),
          pl.BlockSpec((2, BLK, 16), lambda i: (0, i, 0)),
      ],
      out_specs=pl.BlockSpec((BLK, 128), lambda i: (i, 0)),
      out_shape=jax.ShapeDtypeStruct((N, 128), jnp.float32),
  )(x, w, deg)


def _tc_mid2(agg, hp, deg, w, b):
  def body(agg_ref, hp_ref, deg_ref, w_ref, b_ref, o_ref):
    dinv = _dinv_of(deg_ref)
    ag = agg_ref[...]
    xv = jnp.maximum(dinv * (ag[0] + ag[1] + hp_ref[...]) + b_ref[...], 0.0)
    h = jnp.dot(xv, w_ref[...], preferred_element_type=jnp.float32) * dinv
    o_ref[0] = h[:, :32]
    o_ref[1] = h[:, 32:]

  return pl.pallas_call(
      body,
      grid=(GRID,),
      in_specs=[
          pl.BlockSpec((2, BLK, 128), lambda i: (0, i, 0)),
          pl.BlockSpec((BLK, 128), lambda i: (i, 0)),
          pl.BlockSpec((2, BLK, 16), lambda i: (0, i, 0)),
          pl.BlockSpec((128, 64), lambda i: (0, 0)),
          pl.BlockSpec((1, 128), lambda i: (0, 0)),
      ],
      out_specs=pl.BlockSpec((2, BLK, 32), lambda i: (0, i, 0)),
      out_shape=jax.ShapeDtypeStruct((2, NPAD, 32), jnp.float32),
  )(agg, hp, deg, w, b)


def _tc_mid3(agg, hp, deg, w, b):
  def body(agg_ref, hp_ref, deg_ref, w_ref, b_ref, o_ref):
    dinv = _dinv_of(deg_ref)
    ag = agg_ref[...]
    hh = hp_ref[...]
    a = jnp.concatenate([ag[0] + hh[0], ag[1] + hh[1]], axis=1)
    xv = jnp.maximum(dinv * a + b_ref[...], 0.0)
    h = jnp.dot(xv, w_ref[...], preferred_element_type=jnp.float32) * dinv
    o_ref[0] = h[:, :16]
    o_ref[1] = h[:, 16:]

  return pl.pallas_call(
      body,
      grid=(GRID,),
      in_specs=[
          pl.BlockSpec((2, BLK, 32), lambda i: (0, i, 0)),
          pl.BlockSpec((2, BLK, 32), lambda i: (0, i, 0)),
          pl.BlockSpec((2, BLK, 16), lambda i: (0, i, 0)),
          pl.BlockSpec((64, 32), lambda i: (0, 0)),
          pl.BlockSpec((1, 64), lambda i: (0, 0)),
      ],
      out_specs=pl.BlockSpec((2, BLK, 16), lambda i: (0, i, 0)),
      out_shape=jax.ShapeDtypeStruct((2, NPAD, 16), jnp.float32),
  )(agg, hp, deg, w, b)


def _tc_final(agg, hp, deg, b3, wl1, bl1, wl2, bl2, wl3, bl3):
  def body(agg_ref, hp_ref, deg_ref, b3_ref, w1_ref, b1_ref, w2_ref, b2_ref,
           w3_ref, b3h_ref, o_ref):
    dinv = _dinv_of(deg_ref)
    ag = agg_ref[...]
    hh = hp_ref[...]
    a = jnp.concatenate([ag[0] + hh[0], ag[1] + hh[1]], axis=1)
    y = jnp.maximum(dinv * a + b3_ref[...], 0.0)
    y = jnp.maximum(jnp.dot(y, w1_ref[...],
                            preferred_element_type=jnp.float32) + b1_ref[...],
                    0.0)
    y = jnp.maximum(jnp.dot(y, w2_ref[...],
                            preferred_element_type=jnp.float32) + b2_ref[...],
                    0.0)
    y = jnp.maximum(jnp.dot(y, w3_ref[...],
                            preferred_element_type=jnp.float32) + b3h_ref[...],
                    0.0)
    o_ref[...] = y

  return pl.pallas_call(
      body,
      grid=(GRID,),
      in_specs=[
          pl.BlockSpec((2, BLK, 16), lambda i: (0, i, 0)),
          pl.BlockSpec((2, BLK, 16), lambda i: (0, i, 0)),
          pl.BlockSpec((2, BLK, 16), lambda i: (0, i, 0)),
          pl.BlockSpec((1, 32), lambda i: (0, 0)),
          pl.BlockSpec((32, 32), lambda i: (0, 0)),
          pl.BlockSpec((1, 32), lambda i: (0, 0)),
          pl.BlockSpec((32, 16), lambda i: (0, 0)),
          pl.BlockSpec((1, 16), lambda i: (0, 0)),
          pl.BlockSpec((16, 8), lambda i: (0, 0)),
          pl.BlockSpec((1, 8), lambda i: (0, 0)),
      ],
      out_specs=pl.BlockSpec((BLK, 8), lambda i: (i, 0)),
      out_shape=jax.ShapeDtypeStruct((N, 8), jnp.float32),
  )(agg, hp, deg, b3, wl1, bl1, wl2, bl2, wl3, bl3)


def _pad2(w, r, c):
  return jnp.zeros((r, c), w.dtype).at[:w.shape[0], :w.shape[1]].set(w)


def _pad1(b, c):
  return jnp.zeros((1, c), b.dtype).at[0, :b.shape[0]].set(b)


def kernel(inputFeatures, edge_index, W1, b1, W2, b2, W3, b3,
           Wl1, bl1, Wl2, bl2, Wl3, bl3):
  src = edge_index[0]
  dst = edge_index[1]
  src1 = jnp.concatenate(
      [src, jnp.zeros((EPAD1 - E,), src.dtype)]).reshape(32, NCH1, CH1)
  dst1 = jnp.concatenate(
      [dst, jnp.full((EPAD1 - E,), DUMMY, dst.dtype)]).reshape(32, NCH1, CH1)
  src2 = jnp.concatenate(
      [src, jnp.zeros((EPAD2 - E,), src.dtype)]).reshape(16, NCH2, CH2)
  dst2 = jnp.concatenate(
      [dst, jnp.full((EPAD2 - E,), DUMMY, dst.dtype)]).reshape(16, NCH2, CH2)
  ones16 = jnp.ones((CH1, 16), jnp.float32)
  z16 = jnp.zeros((ROWS_PT, 16), jnp.float32)
  z128 = jnp.zeros((ROWS_PT, 128), jnp.float32)
  z32 = jnp.zeros((ROWS_PT, 32), jnp.float32)

  deg = _sc_deg()(dst1, ones16, z16)
  hp1 = _tc_first(inputFeatures, _pad2(W1, 128, 128), deg)
  agg1 = _sc_agg_l1()(hp1, src1, dst1, z128)
  hp2 = _tc_mid2(agg1, hp1, deg, _pad2(W2, 128, 64), _pad1(b1, 128))
  agg2 = _sc_agg_cs(32)(hp2, src2, dst2, z32)
  hp3 = _tc_mid3(agg2, hp2, deg, _pad2(W3, 64, 32), _pad1(b2, 64))
  agg3 = _sc_agg_cs(16)(hp3, src2, dst2, z16)
  y = _tc_final(agg3, hp3, deg, _pad1(b3, 32),
                _pad2(Wl1, 32, 32), _pad1(bl1, 32),
                _pad2(Wl2, 32, 16), _pad1(bl2, 16),
                _pad2(Wl3, 16, 8), _pad1(bl3, 8))
  return y[:, :1]


# speedup vs baseline: 27.3645x; 1.4557x over previous
import functools

import jax
import jax.numpy as jnp
from jax import lax
from jax.experimental import pallas as pl
from jax.experimental.pallas import tpu as pltpu
from jax.experimental.pallas import tpu_sc as plsc

N = 10000
E = 320000
NPAD = 10112
ROWS_PT = NPAD // 16
DUMMY = N
PACK = 16384
BLK = 1000
GRID = N // BLK

CH = 128
NCH = 158
EPAD = 16 * NCH * CH

_SC_PARAMS = pltpu.CompilerParams(use_tc_tiling_on_sc=False)


def _mesh():
  return plsc.VectorSubcoreMesh(core_axis_name="c", subcore_axis_name="s")


def _sc_deg():
  @functools.partial(
      pl.kernel,
      mesh=_mesh(),
      out_type=jax.ShapeDtypeStruct((2, NPAD, 16), jnp.float32),
      scratch_types=[
          pltpu.VMEM((NCH // 2, CH), jnp.int32),
          pltpu.VMEM((CH, 16), jnp.float32),
          pltpu.VMEM((1, CH), jnp.int32),
          pltpu.VMEM_SHARED((NPAD, 16), jnp.float32),
      ],
      compiler_params=_SC_PARAMS,
  )
  def k(packed3, ones_hbm, zeros_hbm, out, pk_v, ones_v, dstch, acc):
    c = lax.axis_index("c")
    s = lax.axis_index("s")
    row0 = s * ROWS_PT
    pltpu.sync_copy(packed3.at[s, pl.ds(c * (NCH // 2), NCH // 2)], pk_v)
    pltpu.sync_copy(ones_hbm, ones_v)
    pltpu.sync_copy(zeros_hbm, acc.at[pl.ds(row0, ROWS_PT)])
    plsc.subcore_barrier()

    def step(j, carry):
      for t in range(CH // 16):
        v = pk_v[j, pl.ds(t * 16, 16)]
        dstch[0, pl.ds(t * 16, 16)] = lax.bitwise_and(v, PACK - 1)
      pltpu.sync_copy(ones_v, acc.at[dstch.at[0]], add=True)
      return carry

    lax.fori_loop(0, NCH // 2, step, 0)
    plsc.subcore_barrier()
    pltpu.sync_copy(acc.at[pl.ds(row0, ROWS_PT)],
                    out.at[c, pl.ds(row0, ROWS_PT)])

  return k


def _sc_agg(hdp):
  @functools.partial(
      pl.kernel,
      mesh=_mesh(),
      out_type=jax.ShapeDtypeStruct((2, NPAD, hdp), jnp.float32),
      scratch_types=[
          pltpu.VMEM((NCH, CH), jnp.int32),
          pltpu.VMEM((2, CH, hdp), jnp.float32),
          pltpu.VMEM((2, CH), jnp.int32),
          pltpu.VMEM((2, CH), jnp.int32),
          pltpu.VMEM_SHARED((NPAD, hdp), jnp.float32),
          pltpu.VMEM_SHARED((NPAD, hdp), jnp.float32),
          pltpu.SemaphoreType.DMA,
          pltpu.SemaphoreType.DMA,
      ],
      compiler_params=_SC_PARAMS,
  )
  def k(hp3d, packed3, zeros_hbm, out, pk_v, rows_v, srcch, dstch,
        hp_sp, acc, sem0, sem1):
    c = lax.axis_index("c")
    s = lax.axis_index("s")
    row0 = s * ROWS_PT
    pltpu.sync_copy(packed3.at[s], pk_v)
    pltpu.sync_copy(hp3d.at[c, pl.ds(row0, ROWS_PT)],
                    hp_sp.at[pl.ds(row0, ROWS_PT)])
    pltpu.sync_copy(zeros_hbm, acc.at[pl.ds(row0, ROWS_PT)])
    plsc.subcore_barrier()

    sems = (sem0, sem1)

    def unpack(j, b):
      for t in range(CH // 16):
        v = pk_v[j, pl.ds(t * 16, 16)]
        srcch[b, pl.ds(t * 16, 16)] = lax.shift_right_logical(v, 14)
        dstch[b, pl.ds(t * 16, 16)] = lax.bitwise_and(v, PACK - 1)

    def fire(b):
      return pltpu.make_async_copy(hp_sp.at[srcch.at[b]], rows_v.at[b],
                                   sems[b])

    unpack(0, 0)
    fire(0).start()
    unpack(1, 1)
    fire(1).start()

    def step(jj, carry):
      for b in range(2):
        j = jj * 2 + b
        fire(b).wait()
        pltpu.sync_copy(rows_v.at[b], acc.at[dstch.at[b]], add=True)
        unpack(j + 2, b)
        fire(b).start()
      return carry

    lax.fori_loop(0, NCH // 2 - 1, step, 0)
    for b in range(2):
      fire(b).wait()
      pltpu.sync_copy(rows_v.at[b], acc.at[dstch.at[b]], add=True)

    plsc.subcore_barrier()
    pltpu.sync_copy(acc.at[pl.ds(row0, ROWS_PT)],
                    out.at[c, pl.ds(row0, ROWS_PT)])

  return k


def _dinv_of(deg_ref):
  d = deg_ref[...]
  return lax.rsqrt(1.0 + d[0, :, :1] + d[1, :, :1])


def _tc_first(x, w, deg):
  def body(x_ref, w_ref, deg_ref, o_ref):
    dinv = _dinv_of(deg_ref)
    h = jnp.dot(x_ref[...], w_ref[...],
                preferred_element_type=jnp.float32) * dinv
    o_ref[0] = h[:, :64]
    o_ref[1] = h[:, 64:]

  return pl.pallas_call(
      body,
      grid=(GRID,),
      in_specs=[
          pl.BlockSpec((BLK, 128), lambda i: (i, 0)),
          pl.BlockSpec((128, 128), lambda i: (0, 0)),
          pl.BlockSpec((2, BLK, 16), lambda i: (0, i, 0)),
      ],
      out_specs=pl.BlockSpec((2, BLK, 64), lambda i: (0, i, 0)),
      out_shape=jax.ShapeDtypeStruct((2, NPAD, 64), jnp.float32),
  )(x, w, deg)


def _tc_mid(agg, hp, deg, w, b, hin, hout):
  def body(agg_ref, hp_ref, deg_ref, w_ref, b_ref, o_ref):
    dinv = _dinv_of(deg_ref)
    ag = agg_ref[...]
    hh = hp_ref[...]
    a = jnp.concatenate([ag[0] + hh[0], ag[1] + hh[1]], axis=1)
    xv = jnp.maximum(dinv * a + b_ref[...], 0.0)
    h = jnp.dot(xv, w_ref[...], preferred_element_type=jnp.float32) * dinv
    o_ref[0] = h[:, :hout]
    o_ref[1] = h[:, hout:]

  return pl.pallas_call(
      body,
      grid=(GRID,),
      in_specs=[
          pl.BlockSpec((2, BLK, hin), lambda i: (0, i, 0)),
          pl.BlockSpec((2, BLK, hin), lambda i: (0, i, 0)),
          pl.BlockSpec((2, BLK, 16), lambda i: (0, i, 0)),
          pl.BlockSpec((2 * hin, 2 * hout), lambda i: (0, 0)),
          pl.BlockSpec((1, 2 * hin), lambda i: (0, 0)),
      ],
      out_specs=pl.BlockSpec((2, BLK, hout), lambda i: (0, i, 0)),
      out_shape=jax.ShapeDtypeStruct((2, NPAD, hout), jnp.float32),
  )(agg, hp, deg, w, b)


def _tc_final(agg, hp, deg, b3, wl1, bl1, wl2, bl2, wl3, bl3):
  def body(agg_ref, hp_ref, deg_ref, b3_ref, w1_ref, b1_ref, w2_ref, b2_ref,
           w3_ref, b3h_ref, o_ref):
    dinv = _dinv_of(deg_ref)
    ag = agg_ref[...]
    hh = hp_ref[...]
    a = jnp.concatenate([ag[0] + hh[0], ag[1] + hh[1]], axis=1)
    y = jnp.maximum(dinv * a + b3_ref[...], 0.0)
    y = jnp.maximum(jnp.dot(y, w1_ref[...],
                            preferred_element_type=jnp.float32) + b1_ref[...],
                    0.0)
    y = jnp.maximum(jnp.dot(y, w2_ref[...],
                            preferred_element_type=jnp.float32) + b2_ref[...],
                    0.0)
    y = jnp.maximum(jnp.dot(y, w3_ref[...],
                            preferred_element_type=jnp.float32) + b3h_ref[...],
                    0.0)
    o_ref[...] = y

  return pl.pallas_call(
      body,
      grid=(GRID,),
      in_specs=[
          pl.BlockSpec((2, BLK, 16), lambda i: (0, i, 0)),
          pl.BlockSpec((2, BLK, 16), lambda i: (0, i, 0)),
          pl.BlockSpec((2, BLK, 16), lambda i: (0, i, 0)),
          pl.BlockSpec((1, 32), lambda i: (0, 0)),
          pl.BlockSpec((32, 32), lambda i: (0, 0)),
          pl.BlockSpec((1, 32), lambda i: (0, 0)),
          pl.BlockSpec((32, 16), lambda i: (0, 0)),
          pl.BlockSpec((1, 16), lambda i: (0, 0)),
          pl.BlockSpec((16, 8), lambda i: (0, 0)),
          pl.BlockSpec((1, 8), lambda i: (0, 0)),
      ],
      out_specs=pl.BlockSpec((BLK, 8), lambda i: (i, 0)),
      out_shape=jax.ShapeDtypeStruct((N, 8), jnp.float32),
  )(agg, hp, deg, b3, wl1, bl1, wl2, bl2, wl3, bl3)


def _pad2(w, r, c):
  return jnp.zeros((r, c), w.dtype).at[:w.shape[0], :w.shape[1]].set(w)


def _pad1(b, c):
  return jnp.zeros((1, c), b.dtype).at[0, :b.shape[0]].set(b)


def kernel(inputFeatures, edge_index, W1, b1, W2, b2, W3, b3,
           Wl1, bl1, Wl2, bl2, Wl3, bl3):
  src = edge_index[0]
  dst = edge_index[1]
  packed = src * PACK + dst
  packed3 = jnp.concatenate(
      [packed,
       jnp.full((EPAD - E,), DUMMY, packed.dtype)]).reshape(16, NCH, CH)
  ones16 = jnp.ones((CH, 16), jnp.float32)
  z16 = jnp.zeros((ROWS_PT, 16), jnp.float32)
  z32 = jnp.zeros((ROWS_PT, 32), jnp.float32)
  z64 = jnp.zeros((ROWS_PT, 64), jnp.float32)

  deg = _sc_deg()(packed3, ones16, z16)
  hp1 = _tc_first(inputFeatures, _pad2(W1, 128, 128), deg)
  agg1 = _sc_agg(64)(hp1, packed3, z64)
  hp2 = _tc_mid(agg1, hp1, deg, _pad2(W2, 128, 64), _pad1(b1, 128), 64, 32)
  agg2 = _sc_agg(32)(hp2, packed3, z32)
  hp3 = _tc_mid(agg2, hp2, deg, _pad2(W3, 64, 32), _pad1(b2, 64), 32, 16)
  agg3 = _sc_agg(16)(hp3, packed3, z16)
  y = _tc_final(agg3, hp3, deg, _pad1(b3, 32),
                _pad2(Wl1, 32, 32), _pad1(bl1, 32),
                _pad2(Wl2, 32, 16), _pad1(bl2, 16),
                _pad2(Wl3, 16, 8), _pad1(bl3, 8))
  return y[:, :1]


# speedup vs baseline: 31.2565x; 1.1422x over previous
import functools

import jax
import jax.numpy as jnp
from jax import lax
from jax.experimental import pallas as pl
from jax.experimental.pallas import tpu as pltpu
from jax.experimental.pallas import tpu_sc as plsc

N = 10000
E = 320000
NPAD = 10112
ROWS_PT = NPAD // 16
DUMMY = N
PACK = 16384
BLK = 1000
GRID = N // BLK

CH = 128
NCH = 158
EPAD = 16 * NCH * CH

_SC_PARAMS = pltpu.CompilerParams(use_tc_tiling_on_sc=False)


def _mesh():
  return plsc.VectorSubcoreMesh(core_axis_name="c", subcore_axis_name="s")


def _sc_deg():
  @functools.partial(
      pl.kernel,
      mesh=_mesh(),
      out_type=jax.ShapeDtypeStruct((NPAD, 128), jnp.float32),
      scratch_types=[
          pltpu.VMEM((NCH // 2, CH), jnp.int32),
          pltpu.VMEM((CH, 16), jnp.float32),
          pltpu.VMEM((1, CH), jnp.int32),
          pltpu.VMEM_SHARED((NPAD, 16), jnp.float32),
      ],
      compiler_params=_SC_PARAMS,
  )
  def k(packed3, ones_hbm, zeros_hbm, out, pk_v, ones_v, dstch, acc):
    c = lax.axis_index("c")
    s = lax.axis_index("s")
    row0 = s * ROWS_PT
    pltpu.sync_copy(packed3.at[s, pl.ds(c * (NCH // 2), NCH // 2)], pk_v)
    pltpu.sync_copy(ones_hbm, ones_v)
    pltpu.sync_copy(zeros_hbm, acc.at[pl.ds(row0, ROWS_PT)])
    plsc.subcore_barrier()

    def step(j, carry):
      for t in range(CH // 16):
        v = pk_v[j, pl.ds(t * 16, 16)]
        dstch[0, pl.ds(t * 16, 16)] = lax.bitwise_and(v, PACK - 1)
      pltpu.sync_copy(ones_v, acc.at[dstch.at[0]], add=True)
      return carry

    lax.fori_loop(0, NCH // 2, step, 0)
    plsc.subcore_barrier()
    pltpu.sync_copy(acc.at[pl.ds(row0, ROWS_PT)],
                    out.at[pl.ds(row0, ROWS_PT), pl.ds(c * 16, 16)])

  return k


def _sc_agg(hdp):
  @functools.partial(
      pl.kernel,
      mesh=_mesh(),
      out_type=jax.ShapeDtypeStruct((NPAD, 128), jnp.float32),
      scratch_types=[
          pltpu.VMEM((NCH, CH), jnp.int32),
          pltpu.VMEM((2, CH, hdp), jnp.float32),
          pltpu.VMEM((2, CH), jnp.int32),
          pltpu.VMEM((2, CH), jnp.int32),
          pltpu.VMEM_SHARED((NPAD, hdp), jnp.float32),
          pltpu.VMEM_SHARED((NPAD, hdp), jnp.float32),
          pltpu.SemaphoreType.DMA,
          pltpu.SemaphoreType.DMA,
      ],
      compiler_params=_SC_PARAMS,
  )
  def k(hp, packed3, zeros_hbm, out, pk_v, rows_v, srcch, dstch,
        hp_sp, acc, sem0, sem1):
    c = lax.axis_index("c")
    s = lax.axis_index("s")
    row0 = s * ROWS_PT
    pltpu.sync_copy(packed3.at[s], pk_v)
    pltpu.sync_copy(hp.at[pl.ds(row0, ROWS_PT), pl.ds(c * hdp, hdp)],
                    hp_sp.at[pl.ds(row0, ROWS_PT)])
    pltpu.sync_copy(zeros_hbm, acc.at[pl.ds(row0, ROWS_PT)])
    plsc.subcore_barrier()

    sems = (sem0, sem1)

    def unpack(j, b):
      for t in range(CH // 16):
        v = pk_v[j, pl.ds(t * 16, 16)]
        srcch[b, pl.ds(t * 16, 16)] = lax.shift_right_logical(v, 14)
        dstch[b, pl.ds(t * 16, 16)] = lax.bitwise_and(v, PACK - 1)

    def fire(b):
      return pltpu.make_async_copy(hp_sp.at[srcch.at[b]], rows_v.at[b],
                                   sems[b])

    unpack(0, 0)
    fire(0).start()
    unpack(1, 1)
    fire(1).start()

    def step(jj, carry):
      for b in range(2):
        j = jj * 2 + b
        fire(b).wait()
        pltpu.sync_copy(rows_v.at[b], acc.at[dstch.at[b]], add=True)
        unpack(j + 2, b)
        fire(b).start()
      return carry

    lax.fori_loop(0, NCH // 2 - 1, step, 0)
    for b in range(2):
      fire(b).wait()
      pltpu.sync_copy(rows_v.at[b], acc.at[dstch.at[b]], add=True)

    plsc.subcore_barrier()
    pltpu.sync_copy(acc.at[pl.ds(row0, ROWS_PT)],
                    out.at[pl.ds(row0, ROWS_PT), pl.ds(c * hdp, hdp)])

  return k


def _dinv_of(deg_ref):
  d = deg_ref[...]
  return lax.rsqrt(1.0 + d[:, :1] + d[:, 16:17])


def _tc_first(x, w, deg):
  def body(x_ref, w_ref, deg_ref, o_ref):
    dinv = _dinv_of(deg_ref)
    o_ref[...] = jnp.dot(x_ref[...], w_ref[...],
                         preferred_element_type=jnp.float32) * dinv

  return pl.pallas_call(
      body,
      grid=(GRID,),
      in_specs=[
          pl.BlockSpec((BLK, 128), lambda i: (i, 0)),
          pl.BlockSpec((128, 128), lambda i: (0, 0)),
          pl.BlockSpec((BLK, 128), lambda i: (i, 0)),
      ],
      out_specs=pl.BlockSpec((BLK, 128), lambda i: (i, 0)),
      out_shape=jax.ShapeDtypeStruct((NPAD, 128), jnp.float32),
  )(x, w, deg)


def _tc_mid(agg, hp, deg, w, b, dpin, dpout):
  def body(agg_ref, hp_ref, deg_ref, w_ref, b_ref, o_ref):
    dinv = _dinv_of(deg_ref)
    a = (agg_ref[...] + hp_ref[...])[:, :dpin]
    xv = jnp.maximum(dinv * a + b_ref[...], 0.0)
    h = jnp.dot(xv, w_ref[...], preferred_element_type=jnp.float32) * dinv
    o_ref[...] = jnp.concatenate(
        [h, jnp.zeros((BLK, 128 - dpout), jnp.float32)], axis=1)

  return pl.pallas_call(
      body,
      grid=(GRID,),
      in_specs=[
          pl.BlockSpec((BLK, 128), lambda i: (i, 0)),
          pl.BlockSpec((BLK, 128), lambda i: (i, 0)),
          pl.BlockSpec((BLK, 128), lambda i: (i, 0)),
          pl.BlockSpec((dpin, dpout), lambda i: (0, 0)),
          pl.BlockSpec((1, dpin), lambda i: (0, 0)),
      ],
      out_specs=pl.BlockSpec((BLK, 128), lambda i: (i, 0)),
      out_shape=jax.ShapeDtypeStruct((NPAD, 128), jnp.float32),
  )(agg, hp, deg, w, b)


def _tc_final(agg, hp, deg, b3, wl1, bl1, wl2, bl2, wl3, bl3):
  def body(agg_ref, hp_ref, deg_ref, b3_ref, w1_ref, b1_ref, w2_ref, b2_ref,
           w3_ref, b3h_ref, o_ref):
    dinv = _dinv_of(deg_ref)
    a = (agg_ref[...] + hp_ref[...])[:, :32]
    y = jnp.maximum(dinv * a + b3_ref[...], 0.0)
    y = jnp.maximum(jnp.dot(y, w1_ref[...],
                            preferred_element_type=jnp.float32) + b1_ref[...],
                    0.0)
    y = jnp.maximum(jnp.dot(y, w2_ref[...],
                            preferred_element_type=jnp.float32) + b2_ref[...],
                    0.0)
    y = jnp.maximum(jnp.dot(y, w3_ref[...],
                            preferred_element_type=jnp.float32) + b3h_ref[...],
                    0.0)
    o_ref[...] = y

  return pl.pallas_call(
      body,
      grid=(GRID,),
      in_specs=[
          pl.BlockSpec((BLK, 128), lambda i: (i, 0)),
          pl.BlockSpec((BLK, 128), lambda i: (i, 0)),
          pl.BlockSpec((BLK, 128), lambda i: (i, 0)),
          pl.BlockSpec((1, 32), lambda i: (0, 0)),
          pl.BlockSpec((32, 32), lambda i: (0, 0)),
          pl.BlockSpec((1, 32), lambda i: (0, 0)),
          pl.BlockSpec((32, 16), lambda i: (0, 0)),
          pl.BlockSpec((1, 16), lambda i: (0, 0)),
          pl.BlockSpec((16, 8), lambda i: (0, 0)),
          pl.BlockSpec((1, 8), lambda i: (0, 0)),
      ],
      out_specs=pl.BlockSpec((BLK, 8), lambda i: (i, 0)),
      out_shape=jax.ShapeDtypeStruct((N, 8), jnp.float32),
  )(agg, hp, deg, b3, wl1, bl1, wl2, bl2, wl3, bl3)


def _pad2(w, r, c):
  return jnp.zeros((r, c), w.dtype).at[:w.shape[0], :w.shape[1]].set(w)


def _pad1(b, c):
  return jnp.zeros((1, c), b.dtype).at[0, :b.shape[0]].set(b)


def kernel(inputFeatures, edge_index, W1, b1, W2, b2, W3, b3,
           Wl1, bl1, Wl2, bl2, Wl3, bl3):
  src = edge_index[0]
  dst = edge_index[1]
  packed = src * PACK + dst
  packed3 = jnp.concatenate(
      [packed,
       jnp.full((EPAD - E,), DUMMY, packed.dtype)]).reshape(16, NCH, CH)
  ones16 = jnp.ones((CH, 16), jnp.float32)
  z16 = jnp.zeros((ROWS_PT, 16), jnp.float32)
  z32 = jnp.zeros((ROWS_PT, 32), jnp.float32)
  z64 = jnp.zeros((ROWS_PT, 64), jnp.float32)

  deg = _sc_deg()(packed3, ones16, z16)
  hp1 = _tc_first(inputFeatures, _pad2(W1, 128, 128), deg)
  agg1 = _sc_agg(64)(hp1, packed3, z64)
  hp2 = _tc_mid(agg1, hp1, deg, _pad2(W2, 128, 64), _pad1(b1, 128), 128, 64)
  agg2 = _sc_agg(32)(hp2, packed3, z32)
  hp3 = _tc_mid(agg2, hp2, deg, _pad2(W3, 64, 32), _pad1(b2, 64), 64, 32)
  agg3 = _sc_agg(16)(hp3, packed3, z16)
  y = _tc_final(agg3, hp3, deg, _pad1(b3, 32),
                _pad2(Wl1, 32, 32), _pad1(bl1, 32),
                _pad2(Wl2, 32, 16), _pad1(bl2, 16),
                _pad2(Wl3, 16, 8), _pad1(bl3, 8))
  return y[:, :1]


# speedup vs baseline: 31.9778x; 1.0231x over previous
import functools

import jax
import jax.numpy as jnp
from jax import lax
from jax.experimental import pallas as pl
from jax.experimental.pallas import tpu as pltpu
from jax.experimental.pallas import tpu_sc as plsc

N = 10000
E = 320000
NPAD = 10112
ROWS_PT = NPAD // 16
DUMMY = N
PACK = 16384
BLK = 1000
GRID = N // BLK

CH = 128
NCH = 160
EPAD = 16 * NCH * CH

_SC_PARAMS = pltpu.CompilerParams(use_tc_tiling_on_sc=False)


def _mesh():
  return plsc.VectorSubcoreMesh(core_axis_name="c", subcore_axis_name="s")


def _sc_deg():
  @functools.partial(
      pl.kernel,
      mesh=_mesh(),
      out_type=jax.ShapeDtypeStruct((NPAD, 128), jnp.float32),
      scratch_types=[
          pltpu.VMEM((NCH // 2, CH), jnp.int32),
          pltpu.VMEM((CH, 16), jnp.float32),
          pltpu.VMEM((CH, 16), jnp.float32),
          pltpu.VMEM((1, CH), jnp.int32),
          pltpu.VMEM_SHARED((NPAD, 16), jnp.float32),
      ],
      compiler_params=_SC_PARAMS,
  )
  def k(packed3, out, pk_v, ones_v, zbuf, dstch, acc):
    c = lax.axis_index("c")
    s = lax.axis_index("s")
    row0 = s * ROWS_PT
    pltpu.sync_copy(packed3.at[s, pl.ds(c * (NCH // 2), NCH // 2)], pk_v)

    def fill(i, carry):
      ones_v[i, :] = jnp.ones((16,), jnp.float32)
      zbuf[i, :] = jnp.zeros((16,), jnp.float32)
      return carry

    lax.fori_loop(0, CH, fill, 0)
    for kk in range(4):
      pltpu.sync_copy(zbuf, acc.at[pl.ds(row0 + kk * CH, CH)])
    pltpu.sync_copy(zbuf.at[pl.ds(0, ROWS_PT - 4 * CH)],
                    acc.at[pl.ds(row0 + 4 * CH, ROWS_PT - 4 * CH)])
    plsc.subcore_barrier()

    def step(j, carry):
      for t in range(CH // 16):
        v = pk_v[j, pl.ds(t * 16, 16)]
        dstch[0, pl.ds(t * 16, 16)] = lax.bitwise_and(v, PACK - 1)
      pltpu.sync_copy(ones_v, acc.at[dstch.at[0]], add=True)
      return carry

    lax.fori_loop(0, NCH // 2, step, 0)
    plsc.subcore_barrier()
    pltpu.sync_copy(acc.at[pl.ds(row0, ROWS_PT)],
                    out.at[pl.ds(row0, ROWS_PT), pl.ds(c * 16, 16)])

  return k


def _sc_agg(hdp):
  @functools.partial(
      pl.kernel,
      mesh=_mesh(),
      out_type=jax.ShapeDtypeStruct((NPAD, 128), jnp.float32),
      scratch_types=[
          pltpu.VMEM((NCH, CH), jnp.int32),
          pltpu.VMEM((2, CH, hdp), jnp.float32),
          pltpu.VMEM((4, CH), jnp.int32),
          pltpu.VMEM((4, CH), jnp.int32),
          pltpu.VMEM_SHARED((NPAD, hdp), jnp.float32),
          pltpu.VMEM_SHARED((NPAD, hdp), jnp.float32),
          pltpu.SemaphoreType.DMA,
          pltpu.SemaphoreType.DMA,
      ],
      compiler_params=_SC_PARAMS,
  )
  def k(hp, packed3, out, pk_v, rows_v, srcch, dstch, hp_sp, acc,
        sem0, sem1):
    c = lax.axis_index("c")
    s = lax.axis_index("s")
    row0 = s * ROWS_PT
    pltpu.sync_copy(packed3.at[s], pk_v)
    pltpu.sync_copy(hp.at[pl.ds(row0, ROWS_PT), pl.ds(c * hdp, hdp)],
                    hp_sp.at[pl.ds(row0, ROWS_PT)])

    def fill(i, carry):
      for t in range(hdp // 16):
        rows_v[0, i, pl.ds(t * 16, 16)] = jnp.zeros((16,), jnp.float32)
      return carry

    lax.fori_loop(0, CH, fill, 0)
    for kk in range(4):
      pltpu.sync_copy(rows_v.at[0], acc.at[pl.ds(row0 + kk * CH, CH)])
    pltpu.sync_copy(rows_v.at[0, pl.ds(0, ROWS_PT - 4 * CH)],
                    acc.at[pl.ds(row0 + 4 * CH, ROWS_PT - 4 * CH)])
    plsc.subcore_barrier()

    sems = (sem0, sem1)

    def unpack(j, u):
      for t in range(CH // 16):
        v = pk_v[j, pl.ds(t * 16, 16)]
        srcch[u, pl.ds(t * 16, 16)] = lax.shift_right_logical(v, 14)
        dstch[u, pl.ds(t * 16, 16)] = lax.bitwise_and(v, PACK - 1)

    def fire(u):
      b = u % 2
      return pltpu.make_async_copy(hp_sp.at[srcch.at[u]], rows_v.at[b],
                                   sems[b])

    unpack(0, 0)
    fire(0).start()
    unpack(1, 1)
    fire(1).start()

    def step(jj, carry):
      for q in range(4):
        j = jj * 4 + q
        fire(q).wait()
        unpack(j + 2, (q + 2) % 4)
        pltpu.sync_copy(rows_v.at[q % 2], acc.at[dstch.at[q]], add=True)
        fire((q + 2) % 4).start()
      return carry

    lax.fori_loop(0, NCH // 4 - 1, step, 0)
    for q in range(4):
      fire(q).wait()
      pltpu.sync_copy(rows_v.at[q % 2], acc.at[dstch.at[q]], add=True)
      if q < 2:
        unpack(NCH - 2 + q, (q + 2) % 4)
        fire((q + 2) % 4).start()

    plsc.subcore_barrier()
    pltpu.sync_copy(acc.at[pl.ds(row0, ROWS_PT)],
                    out.at[pl.ds(row0, ROWS_PT), pl.ds(c * hdp, hdp)])

  return k


def _dinv_of(deg_ref):
  d = deg_ref[...]
  return lax.rsqrt(1.0 + d[:, :1] + d[:, 16:17])


def _tc_first(x, w, deg):
  def body(x_ref, w_ref, deg_ref, o_ref):
    dinv = _dinv_of(deg_ref)
    o_ref[...] = jnp.dot(x_ref[...], w_ref[...],
                         preferred_element_type=jnp.float32) * dinv

  return pl.pallas_call(
      body,
      grid=(GRID,),
      in_specs=[
          pl.BlockSpec((BLK, 128), lambda i: (i, 0)),
          pl.BlockSpec((128, 128), lambda i: (0, 0)),
          pl.BlockSpec((BLK, 128), lambda i: (i, 0)),
      ],
      out_specs=pl.BlockSpec((BLK, 128), lambda i: (i, 0)),
      out_shape=jax.ShapeDtypeStruct((NPAD, 128), jnp.float32),
  )(x, w, deg)


def _tc_mid(agg, hp, deg, w, b, dpin, dpout):
  def body(agg_ref, hp_ref, deg_ref, w_ref, b_ref, o_ref):
    dinv = _dinv_of(deg_ref)
    a = (agg_ref[...] + hp_ref[...])[:, :dpin]
    xv = jnp.maximum(dinv * a + b_ref[...], 0.0)
    h = jnp.dot(xv, w_ref[...], preferred_element_type=jnp.float32) * dinv
    o_ref[...] = jnp.concatenate(
        [h, jnp.zeros((BLK, 128 - dpout), jnp.float32)], axis=1)

  return pl.pallas_call(
      body,
      grid=(GRID,),
      in_specs=[
          pl.BlockSpec((BLK, 128), lambda i: (i, 0)),
          pl.BlockSpec((BLK, 128), lambda i: (i, 0)),
          pl.BlockSpec((BLK, 128), lambda i: (i, 0)),
          pl.BlockSpec((dpin, dpout), lambda i: (0, 0)),
          pl.BlockSpec((1, dpin), lambda i: (0, 0)),
      ],
      out_specs=pl.BlockSpec((BLK, 128), lambda i: (i, 0)),
      out_shape=jax.ShapeDtypeStruct((NPAD, 128), jnp.float32),
  )(agg, hp, deg, w, b)


def _tc_final(agg, hp, deg, b3, wl1, bl1, wl2, bl2, wl3, bl3):
  def body(agg_ref, hp_ref, deg_ref, b3_ref, w1_ref, b1_ref, w2_ref, b2_ref,
           w3_ref, b3h_ref, o_ref):
    dinv = _dinv_of(deg_ref)
    a = (agg_ref[...] + hp_ref[...])[:, :32]
    y = jnp.maximum(dinv * a + b3_ref[...], 0.0)
    y = jnp.maximum(jnp.dot(y, w1_ref[...],
                            preferred_element_type=jnp.float32) + b1_ref[...],
                    0.0)
    y = jnp.maximum(jnp.dot(y, w2_ref[...],
                            preferred_element_type=jnp.float32) + b2_ref[...],
                    0.0)
    y = jnp.maximum(jnp.dot(y, w3_ref[...],
                            preferred_element_type=jnp.float32) + b3h_ref[...],
                    0.0)
    o_ref[...] = y

  return pl.pallas_call(
      body,
      grid=(GRID,),
      in_specs=[
          pl.BlockSpec((BLK, 128), lambda i: (i, 0)),
          pl.BlockSpec((BLK, 128), lambda i: (i, 0)),
          pl.BlockSpec((BLK, 128), lambda i: (i, 0)),
          pl.BlockSpec((1, 32), lambda i: (0, 0)),
          pl.BlockSpec((32, 32), lambda i: (0, 0)),
          pl.BlockSpec((1, 32), lambda i: (0, 0)),
          pl.BlockSpec((32, 16), lambda i: (0, 0)),
          pl.BlockSpec((1, 16), lambda i: (0, 0)),
          pl.BlockSpec((16, 8), lambda i: (0, 0)),
          pl.BlockSpec((1, 8), lambda i: (0, 0)),
      ],
      out_specs=pl.BlockSpec((BLK, 8), lambda i: (i, 0)),
      out_shape=jax.ShapeDtypeStruct((N, 8), jnp.float32),
  )(agg, hp, deg, b3, wl1, bl1, wl2, bl2, wl3, bl3)


def _pad2(w, r, c):
  return jnp.zeros((r, c), w.dtype).at[:w.shape[0], :w.shape[1]].set(w)


def _pad1(b, c):
  return jnp.zeros((1, c), b.dtype).at[0, :b.shape[0]].set(b)


def kernel(inputFeatures, edge_index, W1, b1, W2, b2, W3, b3,
           Wl1, bl1, Wl2, bl2, Wl3, bl3):
  src = edge_index[0]
  dst = edge_index[1]
  packed = src * PACK + dst
  packed3 = jnp.concatenate(
      [packed,
       jnp.full((EPAD - E,), DUMMY, packed.dtype)]).reshape(16, NCH, CH)

  deg = _sc_deg()(packed3)
  hp1 = _tc_first(inputFeatures, _pad2(W1, 128, 128), deg)
  agg1 = _sc_agg(64)(hp1, packed3)
  hp2 = _tc_mid(agg1, hp1, deg, _pad2(W2, 128, 64), _pad1(b1, 128), 128, 64)
  agg2 = _sc_agg(32)(hp2, packed3)
  hp3 = _tc_mid(agg2, hp2, deg, _pad2(W3, 64, 32), _pad1(b2, 64), 64, 32)
  agg3 = _sc_agg(16)(hp3, packed3)
  y = _tc_final(agg3, hp3, deg, _pad1(b3, 32),
                _pad2(Wl1, 32, 32), _pad1(bl1, 32),
                _pad2(Wl2, 32, 16), _pad1(bl2, 16),
                _pad2(Wl3, 16, 8), _pad1(bl3, 8))
  return y[:, :1]
